# baseline (speedup 1.0000x reference)
import jax
import jax.numpy as jnp
from jax.experimental import pallas as pl
from jax.experimental.pallas import tpu as pltpu

_TT = 512


def _moe_body(x32_ref, x16_ref, wg_ref, bg_ref, we_ref, be_ref,
              out_ref, scores_ref):
    k = pl.program_id(1)

    @pl.when(k == 0)
    def _gate_and_init():
        logits = jnp.dot(x32_ref[...], wg_ref[...],
                         preferred_element_type=jnp.float32,
                         precision=jax.lax.Precision.HIGHEST)
        logits = logits + bg_ref[...]
        m = jnp.max(logits, axis=-1, keepdims=True)
        e = jnp.exp(logits - m)
        s = e / jnp.sum(e, axis=-1, keepdims=True)
        scores_ref[...] = s
        out_ref[...] = jnp.dot(s, be_ref[...],
                               preferred_element_type=jnp.float32,
                               precision=jax.lax.Precision.HIGHEST)

    y = jnp.dot(x16_ref[...], we_ref[0], preferred_element_type=jnp.float32)
    s_all = scores_ref[...]
    col = jax.lax.broadcasted_iota(jnp.int32, s_all.shape, 1)
    s_k = jnp.sum(jnp.where(col == k, s_all, 0.0), axis=1, keepdims=True)
    out_ref[...] += y * s_k


def kernel(x, Wg, bg, We, be):
    T, D = x.shape
    E = Wg.shape[1]
    x16 = x.astype(jnp.bfloat16)
    We16 = We.astype(jnp.bfloat16)

    grid = (T // _TT, E)
    out, scores = pl.pallas_call(
        _moe_body,
        grid=grid,
        in_specs=[
            pl.BlockSpec((_TT, D), lambda i, k: (i, 0)),
            pl.BlockSpec((_TT, D), lambda i, k: (i, 0)),
            pl.BlockSpec((D, E), lambda i, k: (0, 0)),
            pl.BlockSpec((1, E), lambda i, k: (0, 0)),
            pl.BlockSpec((1, D, D), lambda i, k: (k, 0, 0)),
            pl.BlockSpec((E, D), lambda i, k: (0, 0)),
        ],
        out_specs=[
            pl.BlockSpec((_TT, D), lambda i, k: (i, 0)),
            pl.BlockSpec((_TT, E), lambda i, k: (i, 0)),
        ],
        out_shape=[
            jax.ShapeDtypeStruct((T, D), jnp.float32),
            jax.ShapeDtypeStruct((T, E), jnp.float32),
        ],
        compiler_params=pltpu.CompilerParams(
            dimension_semantics=("parallel", "arbitrary"),
        ),
    )(x, x16, Wg, bg.reshape(1, E), We16, be)
    return out, scores


# speedup vs baseline: 1.1042x; 1.1042x over previous
import jax
import jax.numpy as jnp
from jax.experimental import pallas as pl
from jax.experimental.pallas import tpu as pltpu

_TT = 1024


def _moe_body(x32_ref, x16_ref, wg_ref, bg_ref, we_ref, be_ref,
              out_ref, scores_ref):
    k = pl.program_id(1)

    @pl.when(k == 0)
    def _gate_and_init():
        logits = jnp.dot(x32_ref[...], wg_ref[...],
                         preferred_element_type=jnp.float32,
                         precision=jax.lax.Precision.HIGHEST)
        logits = logits + bg_ref[...]
        m = jnp.max(logits, axis=-1, keepdims=True)
        e = jnp.exp(logits - m)
        s = e / jnp.sum(e, axis=-1, keepdims=True)
        scores_ref[...] = s
        out_ref[...] = jnp.dot(s, be_ref[...],
                               preferred_element_type=jnp.float32,
                               precision=jax.lax.Precision.HIGHEST)

    y = jnp.dot(x16_ref[...], we_ref[0], preferred_element_type=jnp.float32)
    s_all = scores_ref[...]
    col = jax.lax.broadcasted_iota(jnp.int32, s_all.shape, 1)
    s_k = jnp.sum(jnp.where(col == k, s_all, 0.0), axis=1, keepdims=True)
    out_ref[...] += y * s_k


def kernel(x, Wg, bg, We, be):
    T, D = x.shape
    E = Wg.shape[1]
    x16 = x.astype(jnp.bfloat16)
    We16 = We.astype(jnp.bfloat16)

    grid = (T // _TT, E)
    out, scores = pl.pallas_call(
        _moe_body,
        grid=grid,
        in_specs=[
            pl.BlockSpec((_TT, D), lambda i, k: (i, 0)),
            pl.BlockSpec((_TT, D), lambda i, k: (i, 0)),
            pl.BlockSpec((D, E), lambda i, k: (0, 0)),
            pl.BlockSpec((1, E), lambda i, k: (0, 0)),
            pl.BlockSpec((1, D, D), lambda i, k: (k, 0, 0)),
            pl.BlockSpec((E, D), lambda i, k: (0, 0)),
        ],
        out_specs=[
            pl.BlockSpec((_TT, D), lambda i, k: (i, 0)),
            pl.BlockSpec((_TT, E), lambda i, k: (i, 0)),
        ],
        out_shape=[
            jax.ShapeDtypeStruct((T, D), jnp.float32),
            jax.ShapeDtypeStruct((T, E), jnp.float32),
        ],
        compiler_params=pltpu.CompilerParams(
            dimension_semantics=("parallel", "arbitrary"),
        ),
    )(x, x16, Wg, bg.reshape(1, E), We16, be)
    return out, scores


# speedup vs baseline: 1.4132x; 1.2799x over previous
import jax
import jax.numpy as jnp
from jax.experimental import pallas as pl
from jax.experimental.pallas import tpu as pltpu

_TT = 1024


def _moe_body(x32_ref, x16_ref, wg_ref, bg_ref, we_ref, be_ref,
              out_ref, scores_ref):
    k = pl.program_id(1)

    @pl.when(k == 0)
    def _gate():
        logits = jnp.dot(x32_ref[...], wg_ref[...],
                         preferred_element_type=jnp.float32)
        logits = logits + bg_ref[...]
        m = jnp.max(logits, axis=-1, keepdims=True)
        e = jnp.exp(logits - m)
        scores_ref[...] = e / jnp.sum(e, axis=-1, keepdims=True)

    y = jnp.dot(x16_ref[...], we_ref[0], preferred_element_type=jnp.float32)
    s_all = scores_ref[...]
    col = jax.lax.broadcasted_iota(jnp.int32, s_all.shape, 1)
    s_k = jnp.sum(jnp.where(col == k, s_all, 0.0), axis=1, keepdims=True)
    c = (y + be_ref[0]) * s_k

    @pl.when(k == 0)
    def _init():
        out_ref[...] = c

    @pl.when(k > 0)
    def _acc():
        out_ref[...] += c


def kernel(x, Wg, bg, We, be):
    T, D = x.shape
    E = Wg.shape[1]
    x16 = x.astype(jnp.bfloat16)
    We16 = We.astype(jnp.bfloat16)

    grid = (T // _TT, E)
    out, scores = pl.pallas_call(
        _moe_body,
        grid=grid,
        in_specs=[
            pl.BlockSpec((_TT, D), lambda i, k: (i, 0)),
            pl.BlockSpec((_TT, D), lambda i, k: (i, 0)),
            pl.BlockSpec((D, E), lambda i, k: (0, 0)),
            pl.BlockSpec((1, E), lambda i, k: (0, 0)),
            pl.BlockSpec((1, D, D), lambda i, k: (k, 0, 0)),
            pl.BlockSpec((1, 1, D), lambda i, k: (k, 0, 0)),
        ],
        out_specs=[
            pl.BlockSpec((_TT, D), lambda i, k: (i, 0)),
            pl.BlockSpec((_TT, E), lambda i, k: (i, 0)),
        ],
        out_shape=[
            jax.ShapeDtypeStruct((T, D), jnp.float32),
            jax.ShapeDtypeStruct((T, E), jnp.float32),
        ],
        compiler_params=pltpu.CompilerParams(
            dimension_semantics=("parallel", "arbitrary"),
        ),
    )(x, x16, Wg, bg.reshape(1, E), We16, be.reshape(E, 1, D))
    return out, scores


# speedup vs baseline: 1.5835x; 1.1206x over previous
import jax
import jax.numpy as jnp
from jax.experimental import pallas as pl
from jax.experimental.pallas import tpu as pltpu

_TT = 512


def _moe_body(x_ref, wg_ref, bg_ref, wef_ref, bep_ref,
              out_ref, scores_ref, xs_ref):
    D = x_ref.shape[1]
    E = wg_ref.shape[1]

    x = x_ref[...]
    logits = jnp.dot(x, wg_ref[...], preferred_element_type=jnp.float32)
    logits = logits + bg_ref[...]
    m = jnp.max(logits, axis=-1, keepdims=True)
    e = jnp.exp(logits - m)
    s = e / jnp.sum(e, axis=-1, keepdims=True)
    scores_ref[...] = s

    col = jax.lax.broadcasted_iota(jnp.int32, s.shape, 1)
    for k in range(E):
        s_k = jnp.sum(jnp.where(col == k, s, 0.0), axis=1, keepdims=True)
        xs_ref[:, k * D:(k + 1) * D] = (x * s_k).astype(jnp.bfloat16)

    s128 = jnp.concatenate([s] * (128 // E), axis=1).astype(jnp.bfloat16)
    out_ref[...] = (
        jnp.dot(xs_ref[...], wef_ref[...], preferred_element_type=jnp.float32)
        + jnp.dot(s128, bep_ref[...], preferred_element_type=jnp.float32)
    )


def kernel(x, Wg, bg, We, be):
    T, D = x.shape
    E = Wg.shape[1]
    wef = We.reshape(E * D, D).astype(jnp.bfloat16)
    bep = jnp.zeros((128, D), jnp.bfloat16).at[:E].set(be.astype(jnp.bfloat16))

    out, scores = pl.pallas_call(
        _moe_body,
        grid=(T // _TT,),
        in_specs=[
            pl.BlockSpec((_TT, D), lambda i: (i, 0)),
            pl.BlockSpec((D, E), lambda i: (0, 0)),
            pl.BlockSpec((1, E), lambda i: (0, 0)),
            pl.BlockSpec((E * D, D), lambda i: (0, 0)),
            pl.BlockSpec((128, D), lambda i: (0, 0)),
        ],
        out_specs=[
            pl.BlockSpec((_TT, D), lambda i: (i, 0)),
            pl.BlockSpec((_TT, E), lambda i: (i, 0)),
        ],
        out_shape=[
            jax.ShapeDtypeStruct((T, D), jnp.float32),
            jax.ShapeDtypeStruct((T, E), jnp.float32),
        ],
        scratch_shapes=[pltpu.VMEM((_TT, E * D), jnp.bfloat16)],
        compiler_params=pltpu.CompilerParams(
            dimension_semantics=("arbitrary",),
        ),
    )(x, Wg, bg.reshape(1, E), wef, bep)
    return out, scores
